# Initial kernel scaffold; baseline (speedup 1.0000x reference)
import jax
import jax.numpy as jnp
from jax.experimental import pallas as pl


def kernel(x, gate_w, w1, w2, w3):
    raise NotImplementedError("write your pallas kernel here")


# speedup vs baseline: 1.1669x; 1.1669x over previous
import functools

import jax
import jax.numpy as jnp
from jax.experimental import pallas as pl
from jax.experimental.pallas import tpu as pltpu

N, D, E, H, TOPK = 2048, 768, 8, 2048, 2
AUX_COEF = 0.01
TN = 256
NT = N // TN


def _moe_kernel(x_ref, gate_ref, w1_ref, w2_ref, w3_ref,
                out_ref, aux_ref, combine_ref):
    e = pl.program_id(0)
    n = pl.program_id(1)

    @pl.when(jnp.logical_and(e == 0, n == 0))
    def _router():
        xall = x_ref[...]
        logits = jax.lax.dot_general(
            xall, gate_ref[...], (((1,), (1,)), ((), ())),
            preferred_element_type=jnp.float32)
        m = jnp.max(logits, axis=-1, keepdims=True)
        ex = jnp.exp(logits - m)
        p = ex / jnp.sum(ex, axis=-1, keepdims=True)
        tpe = jnp.mean(p, axis=0)
        aux = AUX_COEF * jnp.mean((tpe - 1.0 / E) ** 2)
        aux_ref[...] = aux.reshape(1, 1)
        idx = jax.lax.broadcasted_iota(jnp.int32, p.shape, 1)
        m1 = jnp.max(p, axis=-1, keepdims=True)
        i1 = jnp.min(jnp.where(p == m1, idx, E), axis=-1, keepdims=True)
        f1 = idx == i1
        pw = jnp.where(f1, -jnp.inf, p)
        m2 = jnp.max(pw, axis=-1, keepdims=True)
        i2 = jnp.min(jnp.where(pw == m2, idx, E), axis=-1, keepdims=True)
        f2 = idx == i2
        denom = m1 + m2
        combine_ref[...] = (jnp.where(f1, p, 0.0) + jnp.where(f2, p, 0.0)) / denom

    xb = x_ref[pl.ds(n * TN, TN), :]
    h1 = jax.lax.dot_general(xb, w1_ref[0], (((1,), (1,)), ((), ())),
                             preferred_element_type=jnp.float32)
    h3 = jax.lax.dot_general(xb, w3_ref[0], (((1,), (1,)), ((), ())),
                             preferred_element_type=jnp.float32)
    h = h1 * jax.nn.sigmoid(h1) * h3
    y = jax.lax.dot_general(h, w2_ref[0], (((1,), (1,)), ((), ())),
                            preferred_element_type=jnp.float32)
    cb = combine_ref[pl.ds(n * TN, TN), :]
    lane = jax.lax.broadcasted_iota(jnp.int32, cb.shape, 1)
    cw = jnp.sum(jnp.where(lane == e, cb, 0.0), axis=-1, keepdims=True)
    contrib = cw * y

    @pl.when(e == 0)
    def _init():
        out_ref[pl.ds(n * TN, TN), :] = contrib

    @pl.when(e > 0)
    def _acc():
        out_ref[pl.ds(n * TN, TN), :] += contrib


@functools.partial(jax.jit, static_argnames=())
def kernel(x, gate_w, w1, w2, w3):
    x2 = x.reshape(N, D)
    out, aux = pl.pallas_call(
        _moe_kernel,
        grid=(E, NT),
        in_specs=[
            pl.BlockSpec((N, D), lambda e, n: (0, 0)),
            pl.BlockSpec((E, D), lambda e, n: (0, 0)),
            pl.BlockSpec((1, H, D), lambda e, n: (e, 0, 0)),
            pl.BlockSpec((1, D, H), lambda e, n: (e, 0, 0)),
            pl.BlockSpec((1, H, D), lambda e, n: (e, 0, 0)),
        ],
        out_specs=[
            pl.BlockSpec((N, D), lambda e, n: (0, 0)),
            pl.BlockSpec((1, 1), lambda e, n: (0, 0)),
        ],
        out_shape=[
            jax.ShapeDtypeStruct((N, D), jnp.float32),
            jax.ShapeDtypeStruct((1, 1), jnp.float32),
        ],
        scratch_shapes=[pltpu.VMEM((N, E), jnp.float32)],
    )(x2, gate_w, w1, w2, w3)
    return out.reshape(x.shape), aux.reshape(())


# speedup vs baseline: 1.4676x; 1.2577x over previous
import functools

import jax
import jax.numpy as jnp
from jax.experimental import pallas as pl
from jax.experimental.pallas import tpu as pltpu

N, D, E, H, TOPK = 2048, 768, 8, 2048, 2
AUX_COEF = 0.01
TN = 256
NTILES = 24
NP = NTILES * TN


def _router_kernel(x_ref, gate_ref, pos0_ref, pos1_ref, p0_ref, p1_ref,
                   te_ref, aux_ref):
    lt = jax.lax.dot_general(gate_ref[...], x_ref[...],
                             (((1,), (1,)), ((), ())),
                             preferred_element_type=jnp.float32)
    m = jnp.max(lt, axis=0, keepdims=True)
    ex = jnp.exp(lt - m)
    p = ex / jnp.sum(ex, axis=0, keepdims=True)

    tpe = jnp.mean(p, axis=1, keepdims=True)
    aux = AUX_COEF * jnp.mean((tpe - 1.0 / E) ** 2)
    aux_ref[...] = aux.reshape(1, 1)

    ie = jax.lax.broadcasted_iota(jnp.int32, p.shape, 0)
    m1 = jnp.max(p, axis=0, keepdims=True)
    i1 = jnp.min(jnp.where(p == m1, ie, E), axis=0, keepdims=True)
    f1 = ie == i1
    pw = jnp.where(f1, -jnp.inf, p)
    m2 = jnp.max(pw, axis=0, keepdims=True)
    i2 = jnp.min(jnp.where(pw == m2, ie, E), axis=0, keepdims=True)
    f2 = ie == i2
    denom = m1 + m2
    p0_ref[...] = m1 / denom
    p1_ref[...] = m2 / denom

    oh0 = f1.astype(jnp.float32)
    oh1 = f2.astype(jnp.float32)

    it_r = jax.lax.broadcasted_iota(jnp.int32, (N, N), 0)
    it_c = jax.lax.broadcasted_iota(jnp.int32, (N, N), 1)
    tri = (it_r < it_c).astype(jnp.float32)
    rank0t = jax.lax.dot_general(oh0, tri, (((1,), (0,)), ((), ())),
                                 preferred_element_type=jnp.float32)
    rank1t = jax.lax.dot_general(oh1, tri, (((1,), (0,)), ((), ())),
                                 preferred_element_type=jnp.float32)
    rank0 = jnp.sum(rank0t * oh0, axis=0, keepdims=True)
    rank1 = jnp.sum(rank1t * oh1, axis=0, keepdims=True)

    c0 = jnp.sum(oh0, axis=1, keepdims=True)
    c1 = jnp.sum(oh1, axis=1, keepdims=True)
    ci = (c0 + c1).astype(jnp.int32)
    pc = jnp.right_shift(ci + (TN - 1), 8) << 8
    ie8r = jax.lax.broadcasted_iota(jnp.int32, (E, E), 0)
    ie8c = jax.lax.broadcasted_iota(jnp.int32, (E, E), 1)
    tri8 = (ie8r > ie8c).astype(jnp.float32)
    po = jax.lax.dot_general(tri8, pc.astype(jnp.float32),
                             (((1,), (0,)), ((), ())),
                             preferred_element_type=jnp.float32)

    pos0 = jnp.sum(po * oh0, axis=0, keepdims=True) + rank0
    pos1 = jnp.sum((po + c0) * oh1, axis=0, keepdims=True) + rank1
    pos0_ref[...] = pos0.astype(jnp.int32)
    pos1_ref[...] = pos1.astype(jnp.int32)

    itile = (jax.lax.broadcasted_iota(jnp.int32, (1, NTILES), 1)
             .astype(jnp.float32) * float(TN))
    ge = (po <= itile).astype(jnp.int32)
    te_ref[...] = jnp.sum(ge, axis=0, keepdims=True) - 1


def _group_kernel(te_ref, xg_ref, w1_ref, w2_ref, w3_ref, yg_ref):
    xb = xg_ref[...]
    h1 = jax.lax.dot_general(xb, w1_ref[0], (((1,), (1,)), ((), ())),
                             preferred_element_type=jnp.float32)
    h3 = jax.lax.dot_general(xb, w3_ref[0], (((1,), (1,)), ((), ())),
                             preferred_element_type=jnp.float32)
    h = h1 * jax.nn.sigmoid(h1) * h3
    yg_ref[...] = jax.lax.dot_general(h, w2_ref[0], (((1,), (1,)), ((), ())),
                                      preferred_element_type=jnp.float32)


def kernel(x, gate_w, w1, w2, w3):
    x2 = x.reshape(N, D)

    pos0, pos1, p0, p1, te, aux = pl.pallas_call(
        _router_kernel,
        in_specs=[
            pl.BlockSpec((N, D), lambda: (0, 0)),
            pl.BlockSpec((E, D), lambda: (0, 0)),
        ],
        out_specs=[
            pl.BlockSpec((1, N), lambda: (0, 0)),
            pl.BlockSpec((1, N), lambda: (0, 0)),
            pl.BlockSpec((1, N), lambda: (0, 0)),
            pl.BlockSpec((1, N), lambda: (0, 0)),
            pl.BlockSpec((1, NTILES), lambda: (0, 0)),
            pl.BlockSpec((1, 1), lambda: (0, 0)),
        ],
        out_shape=[
            jax.ShapeDtypeStruct((1, N), jnp.int32),
            jax.ShapeDtypeStruct((1, N), jnp.int32),
            jax.ShapeDtypeStruct((1, N), jnp.float32),
            jax.ShapeDtypeStruct((1, N), jnp.float32),
            jax.ShapeDtypeStruct((1, NTILES), jnp.int32),
            jax.ShapeDtypeStruct((1, 1), jnp.float32),
        ],
    )(x2, gate_w)

    pos0 = pos0.reshape(N)
    pos1 = pos1.reshape(N)

    posr = jnp.concatenate([pos0, pos1])
    xg = jnp.zeros((NP, D), jnp.float32).at[posr].set(
        jnp.concatenate([x2, x2], axis=0))

    yg = pl.pallas_call(
        _group_kernel,
        grid_spec=pltpu.PrefetchScalarGridSpec(
            num_scalar_prefetch=1,
            grid=(NTILES,),
            in_specs=[
                pl.BlockSpec((TN, D), lambda i, te: (i, 0)),
                pl.BlockSpec((1, H, D), lambda i, te: (te[0, i], 0, 0)),
                pl.BlockSpec((1, D, H), lambda i, te: (te[0, i], 0, 0)),
                pl.BlockSpec((1, H, D), lambda i, te: (te[0, i], 0, 0)),
            ],
            out_specs=pl.BlockSpec((TN, D), lambda i, te: (i, 0)),
        ),
        out_shape=jax.ShapeDtypeStruct((NP, D), jnp.float32),
    )(te, xg, w1, w2, w3)

    out = p0.reshape(N, 1) * yg[pos0] + p1.reshape(N, 1) * yg[pos1]
    return out.reshape(x.shape), aux.reshape(())


# speedup vs baseline: 1.5745x; 1.0728x over previous
import functools

import jax
from jax import lax
import jax.numpy as jnp
from jax.experimental import pallas as pl
from jax.experimental.pallas import tpu as pltpu
from jax.experimental.pallas import tpu_sc as plsc

N, D, E, H, TOPK = 2048, 768, 8, 2048, 2
AUX_COEF = 0.01
TN = 256
NTILES = 24
NP = NTILES * TN

NC, NS = 2, 16
NW = NC * NS
AB = (2 * N) // NW
TB = N // NW


def _router_kernel(x_ref, gate_ref, pos0_ref, pos1_ref, p0_ref, p1_ref,
                   te_ref, aux_ref):
    lt = jax.lax.dot_general(gate_ref[...], x_ref[...],
                             (((1,), (1,)), ((), ())),
                             preferred_element_type=jnp.float32)
    m = jnp.max(lt, axis=0, keepdims=True)
    ex = jnp.exp(lt - m)
    p = ex / jnp.sum(ex, axis=0, keepdims=True)

    tpe = jnp.mean(p, axis=1, keepdims=True)
    aux = AUX_COEF * jnp.mean((tpe - 1.0 / E) ** 2)
    aux_ref[...] = aux.reshape(1, 1)

    ie = jax.lax.broadcasted_iota(jnp.int32, p.shape, 0)
    m1 = jnp.max(p, axis=0, keepdims=True)
    i1 = jnp.min(jnp.where(p == m1, ie, E), axis=0, keepdims=True)
    f1 = ie == i1
    pw = jnp.where(f1, -jnp.inf, p)
    m2 = jnp.max(pw, axis=0, keepdims=True)
    i2 = jnp.min(jnp.where(pw == m2, ie, E), axis=0, keepdims=True)
    f2 = ie == i2
    denom = m1 + m2
    p0_ref[...] = m1 / denom
    p1_ref[...] = m2 / denom

    oh0 = f1.astype(jnp.float32)
    oh1 = f2.astype(jnp.float32)

    it_r = jax.lax.broadcasted_iota(jnp.int32, (N, N), 0)
    it_c = jax.lax.broadcasted_iota(jnp.int32, (N, N), 1)
    tri = (it_r < it_c).astype(jnp.float32)
    rank0t = jax.lax.dot_general(oh0, tri, (((1,), (0,)), ((), ())),
                                 preferred_element_type=jnp.float32)
    rank1t = jax.lax.dot_general(oh1, tri, (((1,), (0,)), ((), ())),
                                 preferred_element_type=jnp.float32)
    rank0 = jnp.sum(rank0t * oh0, axis=0, keepdims=True)
    rank1 = jnp.sum(rank1t * oh1, axis=0, keepdims=True)

    c0 = jnp.sum(oh0, axis=1, keepdims=True)
    c1 = jnp.sum(oh1, axis=1, keepdims=True)
    ci = (c0 + c1).astype(jnp.int32)
    pc = jnp.right_shift(ci + (TN - 1), 8) << 8
    ie8r = jax.lax.broadcasted_iota(jnp.int32, (E, E), 0)
    ie8c = jax.lax.broadcasted_iota(jnp.int32, (E, E), 1)
    tri8 = (ie8r > ie8c).astype(jnp.float32)
    po = jax.lax.dot_general(tri8, pc.astype(jnp.float32),
                             (((1,), (0,)), ((), ())),
                             preferred_element_type=jnp.float32)

    pos0 = jnp.sum(po * oh0, axis=0, keepdims=True) + rank0
    pos1 = jnp.sum((po + c0) * oh1, axis=0, keepdims=True) + rank1
    pos0_ref[...] = pos0.astype(jnp.int32)
    pos1_ref[...] = pos1.astype(jnp.int32)

    itile = (jax.lax.broadcasted_iota(jnp.int32, (1, NTILES), 1)
             .astype(jnp.float32) * float(TN))
    ge = (po <= itile).astype(jnp.int32)
    te_ref[...] = jnp.sum(ge, axis=0, keepdims=True) - 1


def _group_kernel(te_ref, xg_ref, w1_ref, w2_ref, w3_ref, yg_ref):
    xb = xg_ref[...]
    h1 = jax.lax.dot_general(xb, w1_ref[0], (((1,), (1,)), ((), ())),
                             preferred_element_type=jnp.float32)
    h3 = jax.lax.dot_general(xb, w3_ref[0], (((1,), (1,)), ((), ())),
                             preferred_element_type=jnp.float32)
    h = h1 * jax.nn.sigmoid(h1) * h3
    yg_ref[...] = jax.lax.dot_general(h, w2_ref[0], (((1,), (1,)), ((), ())),
                                      preferred_element_type=jnp.float32)


def _dispatch_body(x_hbm, posr_hbm, xg_hbm, idx_v, buf_v, sem):
    w = lax.axis_index("s") * NC + lax.axis_index("c")
    tok0 = (w % NS) * AB
    pltpu.sync_copy(posr_hbm.at[w], idx_v)
    pltpu.sync_copy(x_hbm.at[pl.ds(tok0, AB)], buf_v)
    pltpu.async_copy(buf_v, xg_hbm.at[idx_v], sem).wait()


def _dispatch(x2, posr):
    f = functools.partial(
        pl.kernel,
        out_type=jax.ShapeDtypeStruct((NP, D), jnp.float32),
        mesh=plsc.VectorSubcoreMesh(core_axis_name="c", subcore_axis_name="s"),
        scratch_types=[
            pltpu.VMEM((AB,), jnp.int32),
            pltpu.VMEM((AB, D), jnp.float32),
            pltpu.SemaphoreType.DMA,
        ],
    )(_dispatch_body)
    return f(x2, posr)


def _combine_body(yg_hbm, pos0_hbm, pos1_hbm, p0_hbm, p1_hbm, out_hbm,
                  idx0_v, idx1_v, pv0_v, pv1_v, rows0_v, rows1_v, sem):
    w = lax.axis_index("s") * NC + lax.axis_index("c")
    base = w * TB
    pltpu.sync_copy(pos0_hbm.at[pl.ds(base, TB)], idx0_v)
    pltpu.sync_copy(pos1_hbm.at[pl.ds(base, TB)], idx1_v)
    pltpu.sync_copy(p0_hbm.at[pl.ds(base, TB)], pv0_v)
    pltpu.sync_copy(p1_hbm.at[pl.ds(base, TB)], pv1_v)
    cp0 = pltpu.async_copy(yg_hbm.at[idx0_v], rows0_v, sem)
    cp1 = pltpu.async_copy(yg_hbm.at[idx1_v], rows1_v, sem)
    cp0.wait()
    cp1.wait()

    def grp(g, _):
        pv0 = pv0_v[pl.ds(g * 16, 16)]
        pv1 = pv1_v[pl.ds(g * 16, 16)]
        for i in range(16):
            s0 = pv0[i]
            s1 = pv1[i]
            r = g * 16 + i
            for j in range(D // 16):
                sl = pl.ds(j * 16, 16)
                rows0_v[r, sl] = s0 * rows0_v[r, sl] + s1 * rows1_v[r, sl]
        return _

    lax.fori_loop(0, TB // 16, grp, 0)
    pltpu.sync_copy(rows0_v, out_hbm.at[pl.ds(base, TB)])


def _combine(yg, pos0, pos1, p0, p1):
    f = functools.partial(
        pl.kernel,
        out_type=jax.ShapeDtypeStruct((N, D), jnp.float32),
        mesh=plsc.VectorSubcoreMesh(core_axis_name="c", subcore_axis_name="s"),
        scratch_types=[
            pltpu.VMEM((TB,), jnp.int32),
            pltpu.VMEM((TB,), jnp.int32),
            pltpu.VMEM((TB,), jnp.float32),
            pltpu.VMEM((TB,), jnp.float32),
            pltpu.VMEM((TB, D), jnp.float32),
            pltpu.VMEM((TB, D), jnp.float32),
            pltpu.SemaphoreType.DMA,
        ],
    )(_combine_body)
    return f(yg, pos0, pos1, p0, p1)


def kernel(x, gate_w, w1, w2, w3):
    x2 = x.reshape(N, D)

    pos0, pos1, p0, p1, te, aux = pl.pallas_call(
        _router_kernel,
        in_specs=[
            pl.BlockSpec((N, D), lambda: (0, 0)),
            pl.BlockSpec((E, D), lambda: (0, 0)),
        ],
        out_specs=[
            pl.BlockSpec((1, N), lambda: (0, 0)),
            pl.BlockSpec((1, N), lambda: (0, 0)),
            pl.BlockSpec((1, N), lambda: (0, 0)),
            pl.BlockSpec((1, N), lambda: (0, 0)),
            pl.BlockSpec((1, NTILES), lambda: (0, 0)),
            pl.BlockSpec((1, 1), lambda: (0, 0)),
        ],
        out_shape=[
            jax.ShapeDtypeStruct((1, N), jnp.int32),
            jax.ShapeDtypeStruct((1, N), jnp.int32),
            jax.ShapeDtypeStruct((1, N), jnp.float32),
            jax.ShapeDtypeStruct((1, N), jnp.float32),
            jax.ShapeDtypeStruct((1, NTILES), jnp.int32),
            jax.ShapeDtypeStruct((1, 1), jnp.float32),
        ],
    )(x2, gate_w)

    pos0 = pos0.reshape(N)
    pos1 = pos1.reshape(N)

    posr = jnp.concatenate([pos0, pos1]).reshape(NW, AB)
    xg = _dispatch(x2, posr)

    yg = pl.pallas_call(
        _group_kernel,
        grid_spec=pltpu.PrefetchScalarGridSpec(
            num_scalar_prefetch=1,
            grid=(NTILES,),
            in_specs=[
                pl.BlockSpec((TN, D), lambda i, te: (i, 0)),
                pl.BlockSpec((1, H, D), lambda i, te: (te[0, i], 0, 0)),
                pl.BlockSpec((1, D, H), lambda i, te: (te[0, i], 0, 0)),
                pl.BlockSpec((1, H, D), lambda i, te: (te[0, i], 0, 0)),
            ],
            out_specs=pl.BlockSpec((TN, D), lambda i, te: (i, 0)),
        ),
        out_shape=jax.ShapeDtypeStruct((NP, D), jnp.float32),
    )(te, xg, w1, w2, w3)

    out = _combine(yg, pos0, pos1, p0.reshape(N), p1.reshape(N))
    return out.reshape(x.shape), aux.reshape(())
